# baseline (speedup 1.0000x reference)
import functools

import jax
import jax.numpy as jnp
from jax import lax
from jax.experimental import pallas as pl
from jax.experimental.pallas import tpu as pltpu
from jax.experimental.pallas import tpu_sc as plsc

_N = 16384
_D = 784
_K = 10
_NK = _N * _K

_NUM_WORKERS = 32
_CHUNK = _NK // _NUM_WORKERS
_PTS = _CHUNK // _K
_LANES = 16


def _mlp_body(xt_ref, w1_ref, b1_ref, w2_ref, b2_ref, w3_ref, b3_ref,
              wo_ref, bo_ref, out_ref):
    h = lax.dot_general(
        xt_ref[...], w1_ref[...],
        dimension_numbers=(((0,), (0,)), ((), ())),
        preferred_element_type=jnp.float32,
    )
    h = jnp.maximum(h + b1_ref[...], 0.0)
    h = lax.dot_general(
        h, w2_ref[...],
        dimension_numbers=(((1,), (1,)), ((), ())),
        preferred_element_type=jnp.float32,
    )
    h = jnp.maximum(h + b2_ref[...], 0.0)
    h = lax.dot_general(
        h, w3_ref[...],
        dimension_numbers=(((1,), (1,)), ((), ())),
        preferred_element_type=jnp.float32,
    )
    h = jnp.maximum(h + b3_ref[...], 0.0)
    out_ref[...] = (
        lax.dot_general(
            wo_ref[...], h,
            dimension_numbers=(((1,), (1,)), ((), ())),
            preferred_element_type=jnp.float32,
        )
        + bo_ref[...]
    )


def _make_mlp(block_rows):
    grid = (_N // block_rows,)
    fixed = lambda i: (0, 0)
    return pl.pallas_call(
        _mlp_body,
        grid=grid,
        in_specs=[
            pl.BlockSpec((_D, block_rows), lambda i: (0, i)),
            pl.BlockSpec((_D, 128), fixed),
            pl.BlockSpec((1, 128), fixed),
            pl.BlockSpec((64, 128), fixed),
            pl.BlockSpec((1, 64), fixed),
            pl.BlockSpec((32, 64), fixed),
            pl.BlockSpec((1, 32), fixed),
            pl.BlockSpec((2, 32), fixed),
            pl.BlockSpec((2, 1), fixed),
        ],
        out_specs=pl.BlockSpec((2, block_rows), lambda i: (0, i)),
        out_shape=jax.ShapeDtypeStruct((2, _N), jnp.float32),
    )


_mlp = _make_mlp(2048)


_sc_mesh = plsc.VectorSubcoreMesh(core_axis_name="c", subcore_axis_name="s")


@functools.partial(
    pl.kernel,
    mesh=_sc_mesh,
    compiler_params=pltpu.CompilerParams(needs_layout_passes=False),
    out_type=jax.ShapeDtypeStruct((_NK,), jnp.float32),
    scratch_types=[
        pltpu.VMEM((2 * _N,), jnp.float32),
        pltpu.VMEM((_CHUNK,), jnp.int32),
        pltpu.VMEM((_CHUNK,), jnp.float32),
        pltpu.SemaphoreType.DMA,
    ],
)
def _sc_dists(flat_hbm, nidxt_hbm, out_hbm, tab_v, nidx_v, out_v, sem):
    wid = lax.axis_index("s") * 2 + lax.axis_index("c")
    base = wid * _CHUNK
    i0 = wid * _PTS
    copies = [pltpu.async_copy(flat_hbm, tab_v, sem)]
    for k in range(_K):
        copies.append(pltpu.async_copy(
            nidxt_hbm.at[pl.ds(k * _N + i0, _PTS)],
            nidx_v.at[pl.ds(k * _PTS, _PTS)],
            sem,
        ))
    for c in copies:
        c.wait()
    lane = lax.iota(jnp.int32, _LANES)
    oidx0 = lane * _K

    @plsc.parallel_loop(0, _PTS // _LANES, 1, unroll=2)
    def body(j):
        off = j * _LANES
        ax = tab_v[pl.ds(i0 + off, _LANES)]
        ay = tab_v[pl.ds(_N + i0 + off, _LANES)]
        obase = oidx0 + off * _K
        for k in range(_K):
            ni = nidx_v[pl.ds(k * _PTS + off, _LANES)]
            bx = plsc.load_gather(tab_v, [ni])
            by = plsc.load_gather(tab_v, [ni + _N])
            dx = ax - bx
            dy = ay - by
            plsc.store_scatter(out_v, [obase + k], dx * dx + dy * dy)

    pltpu.sync_copy(out_v, out_hbm.at[pl.ds(base, _CHUNK)])


def kernel(data, idxs, W1, b1, W2, b2, W3, b3, Wo, bo):
    projected = _mlp(
        data.T,
        W1, b1.reshape(1, -1),
        W2.T, b2.reshape(1, -1),
        W3.T, b3.reshape(1, -1),
        Wo.T, bo.reshape(-1, 1),
    )
    flat = projected.reshape(-1)
    nidxt = idxs.T.reshape(-1).astype(jnp.int32)
    dists = _sc_dists(flat, nidxt)
    return dists.reshape(-1, 1)


# speedup vs baseline: 1.0233x; 1.0233x over previous
import functools

import jax
import jax.numpy as jnp
from jax import lax
from jax.experimental import pallas as pl
from jax.experimental.pallas import tpu as pltpu
from jax.experimental.pallas import tpu_sc as plsc

_N = 16384
_D = 784
_K = 10
_NK = _N * _K

_NUM_WORKERS = 32
_CHUNK = _NK // _NUM_WORKERS
_PTS = _CHUNK // _K
_LANES = 16


def _mlp_body(xt_ref, w1_ref, b1_ref, w2_ref, b2_ref, w3_ref, b3_ref,
              wo_ref, bo_ref, px_ref, py_ref):
    h = lax.dot_general(
        xt_ref[...], w1_ref[...],
        dimension_numbers=(((0,), (0,)), ((), ())),
        preferred_element_type=jnp.float32,
    )
    h = jnp.maximum(h + b1_ref[...], 0.0)
    h = lax.dot_general(
        h, w2_ref[...],
        dimension_numbers=(((1,), (1,)), ((), ())),
        preferred_element_type=jnp.float32,
    )
    h = jnp.maximum(h + b2_ref[...], 0.0)
    h = lax.dot_general(
        h, w3_ref[...],
        dimension_numbers=(((1,), (1,)), ((), ())),
        preferred_element_type=jnp.float32,
    )
    h = jnp.maximum(h + b3_ref[...], 0.0)
    out = (
        lax.dot_general(
            wo_ref[...], h,
            dimension_numbers=(((1,), (1,)), ((), ())),
            preferred_element_type=jnp.float32,
        )
        + bo_ref[...]
    )
    px_ref[...] = out[0]
    py_ref[...] = out[1]


def _make_mlp(block_rows):
    grid = (_N // block_rows,)
    fixed = lambda i: (0, 0)
    return pl.pallas_call(
        _mlp_body,
        grid=grid,
        in_specs=[
            pl.BlockSpec((_D, block_rows), lambda i: (0, i)),
            pl.BlockSpec((_D, 128), fixed),
            pl.BlockSpec((1, 128), fixed),
            pl.BlockSpec((64, 128), fixed),
            pl.BlockSpec((1, 64), fixed),
            pl.BlockSpec((32, 64), fixed),
            pl.BlockSpec((1, 32), fixed),
            pl.BlockSpec((2, 32), fixed),
            pl.BlockSpec((2, 1), fixed),
        ],
        out_specs=[pl.BlockSpec((block_rows,), lambda i: (i,)),
                   pl.BlockSpec((block_rows,), lambda i: (i,))],
        out_shape=[jax.ShapeDtypeStruct((_N,), jnp.float32),
                   jax.ShapeDtypeStruct((_N,), jnp.float32)],
    )


_mlp = _make_mlp(4096)


_sc_mesh = plsc.VectorSubcoreMesh(core_axis_name="c", subcore_axis_name="s")


@functools.partial(
    pl.kernel,
    mesh=_sc_mesh,
    compiler_params=pltpu.CompilerParams(needs_layout_passes=False),
    out_type=jax.ShapeDtypeStruct((_NK,), jnp.float32),
    scratch_types=[
        pltpu.VMEM((2 * _N,), jnp.float32),
        pltpu.VMEM((_CHUNK,), jnp.int32),
        pltpu.VMEM((_CHUNK,), jnp.float32),
        pltpu.SemaphoreType.DMA,
    ],
)
def _sc_dists(px_hbm, py_hbm, nidxt_hbm, out_hbm, tab_v, nidx_v, out_v, sem):
    wid = lax.axis_index("s") * 2 + lax.axis_index("c")
    base = wid * _CHUNK
    i0 = wid * _PTS
    copies = [pltpu.async_copy(px_hbm, tab_v.at[pl.ds(0, _N)], sem),
              pltpu.async_copy(py_hbm, tab_v.at[pl.ds(_N, _N)], sem)]
    for k in range(_K):
        copies.append(pltpu.async_copy(
            nidxt_hbm.at[pl.ds(k * _N + i0, _PTS)],
            nidx_v.at[pl.ds(k * _PTS, _PTS)],
            sem,
        ))
    for c in copies:
        c.wait()
    lane = lax.iota(jnp.int32, _LANES)
    oidx0 = lane * _K

    @plsc.parallel_loop(0, _PTS // _LANES, 1, unroll=2)
    def body(j):
        off = j * _LANES
        ax = tab_v[pl.ds(i0 + off, _LANES)]
        ay = tab_v[pl.ds(_N + i0 + off, _LANES)]
        obase = oidx0 + off * _K
        for k in range(_K):
            ni = nidx_v[pl.ds(k * _PTS + off, _LANES)]
            bx = plsc.load_gather(tab_v, [ni])
            by = plsc.load_gather(tab_v, [ni + _N])
            dx = ax - bx
            dy = ay - by
            plsc.store_scatter(out_v, [obase + k], dx * dx + dy * dy)

    pltpu.sync_copy(out_v, out_hbm.at[pl.ds(base, _CHUNK)])


def kernel(data, idxs, W1, b1, W2, b2, W3, b3, Wo, bo):
    px, py = _mlp(
        data.T,
        W1, b1.reshape(1, -1),
        W2.T, b2.reshape(1, -1),
        W3.T, b3.reshape(1, -1),
        Wo.T, bo.reshape(-1, 1),
    )
    nidxt = idxs.T.reshape(-1).astype(jnp.int32)
    dists = _sc_dists(px, py, nidxt)
    return dists.reshape(-1, 1)
